# baseline (speedup 1.0000x reference)
import functools

import jax
import jax.numpy as jnp
from jax import lax
from jax.experimental import pallas as pl
from jax.experimental.pallas import tpu as pltpu
from jax.experimental.pallas import tpu_sc as plsc

DIM = 256
L1W = DIM - 4
NB, SEQ = 4096, 50
B = NB * SEQ
NC, NS = 2, 16
NW = NC * NS
CH = 128

M = 2048
TCB = 60
BT = TCB * M
BSC = B - BT
BPW = BSC // NW
NCHUNK = BPW // CH


def _prep_body(l1s_ref, vc_ref, l1t_ref, vt_ref, fused_ref, comb_ref,
               b0_ref, b1_ref, b2_ref):
    l1 = jnp.clip(l1s_ref[...].astype(jnp.int32), 0, 255)
    v = jnp.clip(vc_ref[...].astype(jnp.int32), 0, 3)
    fused_ref[...] = l1 * 4 + v
    t = l1t_ref[...]
    comb_ref[:, :L1W] = jnp.broadcast_to(t[:, None, :], (256, 4, L1W)).reshape(1024, L1W)
    vt = vt_ref[...]
    comb_ref[:, L1W:] = jnp.broadcast_to(vt[None, :, :], (256, 4, 4)).reshape(1024, 4)
    c = comb_ref[...]
    b0 = c.astype(jnp.bfloat16)
    r1 = c - b0.astype(jnp.float32)
    b1 = r1.astype(jnp.bfloat16)
    r2 = r1 - b1.astype(jnp.float32)
    b0_ref[...] = b0
    b1_ref[...] = b1
    b2_ref[...] = r2.astype(jnp.bfloat16)


_prep = pl.pallas_call(
    _prep_body,
    out_shape=[
        jax.ShapeDtypeStruct((NB, SEQ), jnp.int32),
        jax.ShapeDtypeStruct((1024, DIM), jnp.float32),
        jax.ShapeDtypeStruct((1024, DIM), jnp.bfloat16),
        jax.ShapeDtypeStruct((1024, DIM), jnp.bfloat16),
        jax.ShapeDtypeStruct((1024, DIM), jnp.bfloat16),
    ],
)


def _mm_body(idx_ref, b0_ref, b1_ref, b2_ref, _sc_ref, out_ref):
    idx = idx_ref[0, 0, :]
    iota = lax.broadcasted_iota(jnp.int32, (M, 1024), 1)
    oh = (iota == idx[:, None]).astype(jnp.bfloat16)
    acc = jnp.dot(oh, b0_ref[...], preferred_element_type=jnp.float32)
    acc += jnp.dot(oh, b1_ref[...], preferred_element_type=jnp.float32)
    acc += jnp.dot(oh, b2_ref[...], preferred_element_type=jnp.float32)
    out_ref[...] = acc


_mm = pl.pallas_call(
    _mm_body,
    grid=(TCB,),
    in_specs=[
        pl.BlockSpec((1, 1, M), lambda i: (i, 0, 0)),
        pl.BlockSpec((1024, DIM), lambda i: (0, 0)),
        pl.BlockSpec((1024, DIM), lambda i: (0, 0)),
        pl.BlockSpec((1024, DIM), lambda i: (0, 0)),
        pl.BlockSpec(memory_space=pl.ANY),
    ],
    out_specs=pl.BlockSpec((M, DIM), lambda i: (i, 0)),
    out_shape=jax.ShapeDtypeStruct((B, DIM), jnp.float32),
    input_output_aliases={4: 0},
)


@functools.cache
def _make_sc_gather():
    @functools.partial(
        pl.kernel,
        out_type=jax.ShapeDtypeStruct((B, DIM), jnp.float32),
        mesh=plsc.VectorSubcoreMesh(core_axis_name="c", subcore_axis_name="s"),
        scratch_types=[
            pltpu.VMEM((NCHUNK, CH), jnp.int32),
            pltpu.VMEM((CH, DIM), jnp.float32),
            pltpu.VMEM((CH, DIM), jnp.float32),
            pltpu.SemaphoreType.DMA,
            pltpu.SemaphoreType.DMA,
        ],
    )
    def _sc_gather(tbl_hbm, idx_hbm, out_hbm, idx_v, rows0, rows1, sem0, sem1):
        wid = lax.axis_index("s") * NC + lax.axis_index("c")
        base = BT + wid * BPW
        pltpu.sync_copy(idx_hbm.at[wid], idx_v)
        rows = (rows0, rows1)
        sems = (sem0, sem1)

        pltpu.async_copy(tbl_hbm.at[idx_v.at[0]], rows0, sem0)

        def outer(jo, carry):
            for b in range(2):
                j = jo * 2 + b
                nb = 1 - b

                @pl.when(j + 1 < NCHUNK)
                def _():
                    pltpu.async_copy(tbl_hbm.at[idx_v.at[j + 1]], rows[nb], sems[nb])

                pltpu.make_async_copy(tbl_hbm.at[pl.ds(0, CH)], rows[b], sems[b]).wait()
                pltpu.sync_copy(rows[b], out_hbm.at[pl.ds(base + j * CH, CH)])
            return carry

        lax.fori_loop(0, NCHUNK // 2, outer, 0)

    return _sc_gather


def kernel(l1_states, vertex_charges, l1_table, vertex_table):
    fused, comb, b0, b1, b2 = _prep(
        l1_states.astype(jnp.int32),
        vertex_charges.astype(jnp.int32),
        l1_table,
        vertex_table,
    )
    flat = fused.reshape(-1)
    sc_idx = flat[BT:].reshape(NW, NCHUNK, CH)
    sc_full = _make_sc_gather()(comb, sc_idx)
    tc_idx = flat[:BT].reshape(TCB, 1, M)
    out = _mm(tc_idx, b0, b1, b2, sc_full)
    return out.reshape(NB, SEQ, DIM)


# speedup vs baseline: 1.2063x; 1.2063x over previous
import functools

import jax
import jax.numpy as jnp
from jax import lax
from jax.experimental import pallas as pl
from jax.experimental.pallas import tpu as pltpu
from jax.experimental.pallas import tpu_sc as plsc

DIM = 256
L1W = DIM - 4
NB, SEQ = 4096, 50
B = NB * SEQ
NC, NS = 2, 16
NW = NC * NS
BPW = B // NW
CH = 128
NCHUNK = BPW // CH


def _prep_body(l1s_ref, vc_ref, l1t_ref, vt_ref, fused_ref, comb_ref):
    l1 = jnp.clip(l1s_ref[...].astype(jnp.int32), 0, 255)
    v = jnp.clip(vc_ref[...].astype(jnp.int32), 0, 3)
    fused_ref[...] = l1 * 4 + v
    t = l1t_ref[...]
    comb_ref[:, :L1W] = jnp.broadcast_to(t[:, None, :], (256, 4, L1W)).reshape(1024, L1W)
    vt = vt_ref[...]
    comb_ref[:, L1W:] = jnp.broadcast_to(vt[None, :, :], (256, 4, 4)).reshape(1024, 4)


_prep = pl.pallas_call(
    _prep_body,
    out_shape=[
        jax.ShapeDtypeStruct((NB, SEQ), jnp.int32),
        jax.ShapeDtypeStruct((1024, DIM), jnp.float32),
    ],
)


@functools.cache
def _make_sc_gather():
    @functools.partial(
        pl.kernel,
        out_type=jax.ShapeDtypeStruct((B, DIM), jnp.float32),
        mesh=plsc.VectorSubcoreMesh(core_axis_name="c", subcore_axis_name="s"),
        scratch_types=[
            pltpu.VMEM((NCHUNK, CH), jnp.int32),
            pltpu.VMEM((CH, DIM), jnp.float32),
            pltpu.VMEM((CH, DIM), jnp.float32),
            pltpu.SemaphoreType.DMA,
            pltpu.SemaphoreType.DMA,
        ],
    )
    def _sc_gather(tbl_hbm, idx_hbm, out_hbm, idx_v, rows0, rows1, sem0, sem1):
        wid = lax.axis_index("s") * NC + lax.axis_index("c")
        base = wid * BPW
        pltpu.sync_copy(idx_hbm.at[wid], idx_v)
        rows = (rows0, rows1)
        sems = (sem0, sem1)

        pltpu.async_copy(tbl_hbm.at[idx_v.at[0]], rows0, sem0)

        def outer(jo, carry):
            for b in range(2):
                j = jo * 2 + b
                nb = 1 - b

                @pl.when(j + 1 < NCHUNK)
                def _():
                    pltpu.async_copy(tbl_hbm.at[idx_v.at[j + 1]], rows[nb], sems[nb])

                pltpu.make_async_copy(tbl_hbm.at[pl.ds(0, CH)], rows[b], sems[b]).wait()
                pltpu.sync_copy(rows[b], out_hbm.at[pl.ds(base + j * CH, CH)])
            return carry

        lax.fori_loop(0, NCHUNK // 2, outer, 0)

    return _sc_gather


def kernel(l1_states, vertex_charges, l1_table, vertex_table):
    fused, comb = _prep(
        l1_states.astype(jnp.int32),
        vertex_charges.astype(jnp.int32),
        l1_table,
        vertex_table,
    )
    idx3 = fused.reshape(NW, NCHUNK, CH)
    out = _make_sc_gather()(comb, idx3)
    return out.reshape(NB, SEQ, DIM)
